# Initial kernel scaffold; baseline (speedup 1.0000x reference)
import jax
import jax.numpy as jnp
from jax.experimental import pallas as pl


def kernel(x, adj_norm_pos, adj_norm_neg, W_pos, b_pos, W_neg, b_neg, W1, b1, W2, b2, W3, b3, Wd):
    raise NotImplementedError("write your pallas kernel here")


# speedup vs baseline: 1.0183x; 1.0183x over previous
import jax
import jax.numpy as jnp
from jax.experimental import pallas as pl

_N = 10000
_NFEAT = 128
_NHID = 128
_DHID1 = 64

_BI = 200
_BD = 400


def _dot(a, b):
    return jnp.dot(a, b, preferred_element_type=jnp.float32)


def _xw_kernel(x_ref, wp_ref, wn_ref, xwp_ref, xwn_ref):
    x = x_ref[...]
    xwp_ref[...] = _dot(x, wp_ref[...])
    xwn_ref[...] = _dot(x, wn_ref[...])


def _enc_kernel(ap_ref, an_ref, xwp_ref, xwn_ref, bp_ref, bn_ref,
                w1p_ref, w1n_ref, b1_ref, w2_ref, b2_ref, w3_ref, b3_ref,
                wd_ref, h_ref, t_ref):
    zp = jax.nn.relu(_dot(ap_ref[...], xwp_ref[...]) + bp_ref[...])
    zn = jax.nn.relu(_dot(an_ref[...], xwn_ref[...]) + bn_ref[...])
    h1 = jax.nn.relu(_dot(zp, w1p_ref[...]) + _dot(zn, w1n_ref[...])
                     + b1_ref[...])
    h2 = jax.nn.relu(_dot(h1, w2_ref[...]) + b2_ref[...])
    h = _dot(h2, w3_ref[...]) + b3_ref[...]
    h_ref[...] = h
    t_ref[...] = _dot(h, wd_ref[...])


def _dec_kernel(t_ref, h_ref, y_ref):
    y_ref[...] = jax.lax.dot_general(
        t_ref[...], h_ref[...], (((1,), (1,)), ((), ())),
        preferred_element_type=jnp.float32)


def kernel(x, adj_norm_pos, adj_norm_neg, W_pos, b_pos, W_neg, b_neg,
           W1, b1, W2, b2, W3, b3, Wd):
    f32 = jnp.float32

    xwp, xwn = pl.pallas_call(
        _xw_kernel,
        out_shape=[jax.ShapeDtypeStruct((_N, _NHID), f32)] * 2,
    )(x, W_pos, W_neg)

    full = lambda shape: pl.BlockSpec(shape, lambda i: (0, 0))
    h, t = pl.pallas_call(
        _enc_kernel,
        grid=(_N // _BI,),
        in_specs=[
            pl.BlockSpec((_BI, _N), lambda i: (i, 0)),
            pl.BlockSpec((_BI, _N), lambda i: (i, 0)),
            full((_N, _NHID)),
            full((_N, _NHID)),
            full((1, _NHID)),
            full((1, _NHID)),
            full((_NHID, _DHID1)),
            full((_NHID, _DHID1)),
            full((1, _DHID1)),
            full((_DHID1, 2 * _DHID1)),
            full((1, 2 * _DHID1)),
            full((2 * _DHID1, _DHID1)),
            full((1, _DHID1)),
            full((_DHID1, _DHID1)),
        ],
        out_specs=[
            pl.BlockSpec((_BI, _DHID1), lambda i: (i, 0)),
            pl.BlockSpec((_BI, _DHID1), lambda i: (i, 0)),
        ],
        out_shape=[jax.ShapeDtypeStruct((_N, _DHID1), f32)] * 2,
    )(adj_norm_pos, adj_norm_neg, xwp, xwn,
      b_pos.reshape(1, -1), b_neg.reshape(1, -1),
      W1[:_NHID], W1[_NHID:], b1.reshape(1, -1),
      W2, b2.reshape(1, -1), W3, b3.reshape(1, -1), Wd)

    y = pl.pallas_call(
        _dec_kernel,
        grid=(_N // _BD,),
        in_specs=[
            pl.BlockSpec((_BD, _DHID1), lambda i: (i, 0)),
            pl.BlockSpec((_N, _DHID1), lambda i: (0, 0)),
        ],
        out_specs=pl.BlockSpec((_BD, _N), lambda i: (i, 0)),
        out_shape=jax.ShapeDtypeStruct((_N, _N), f32),
    )(t, h)
    return y


# speedup vs baseline: 1.0417x; 1.0230x over previous
import jax
import jax.numpy as jnp
from jax.experimental import pallas as pl
from jax.experimental.pallas import tpu as pltpu

_N = 10000
_NFEAT = 128
_NHID = 128
_DHID1 = 64

_BI = 200
_BD = 400


def _dot(a, b):
    return jnp.dot(a, b, preferred_element_type=jnp.float32)


def _enc_kernel(ap_ref, an_ref, x_ref, wp_ref, wn_ref, bp_ref, bn_ref,
                w1p_ref, w1n_ref, b1_ref, w2_ref, b2_ref, w3_ref, b3_ref,
                wd_ref, h_ref, t_ref, xwp_ref, xwn_ref):
    @pl.when(pl.program_id(0) == 0)
    def _():
        x = x_ref[...]
        xwp_ref[...] = _dot(x, wp_ref[...])
        xwn_ref[...] = _dot(x, wn_ref[...])

    zp = jax.nn.relu(_dot(ap_ref[...], xwp_ref[...]) + bp_ref[...])
    zn = jax.nn.relu(_dot(an_ref[...], xwn_ref[...]) + bn_ref[...])
    h1 = jax.nn.relu(_dot(zp, w1p_ref[...]) + _dot(zn, w1n_ref[...])
                     + b1_ref[...])
    h2 = jax.nn.relu(_dot(h1, w2_ref[...]) + b2_ref[...])
    h = _dot(h2, w3_ref[...]) + b3_ref[...]
    h_ref[...] = h
    t_ref[...] = _dot(h, wd_ref[...])


def _dec_kernel(t_ref, h_ref, y_ref):
    y_ref[...] = jax.lax.dot_general(
        t_ref[...], h_ref[...], (((1,), (1,)), ((), ())),
        preferred_element_type=jnp.float32)


def kernel(x, adj_norm_pos, adj_norm_neg, W_pos, b_pos, W_neg, b_neg,
           W1, b1, W2, b2, W3, b3, Wd):
    f32 = jnp.float32

    full = lambda shape: pl.BlockSpec(shape, lambda i: (0, 0))
    h, t = pl.pallas_call(
        _enc_kernel,
        grid=(_N // _BI,),
        in_specs=[
            pl.BlockSpec((_BI, _N), lambda i: (i, 0)),
            pl.BlockSpec((_BI, _N), lambda i: (i, 0)),
            full((_N, _NFEAT)),
            full((_NFEAT, _NHID)),
            full((_NFEAT, _NHID)),
            full((1, _NHID)),
            full((1, _NHID)),
            full((_NHID, _DHID1)),
            full((_NHID, _DHID1)),
            full((1, _DHID1)),
            full((_DHID1, 2 * _DHID1)),
            full((1, 2 * _DHID1)),
            full((2 * _DHID1, _DHID1)),
            full((1, _DHID1)),
            full((_DHID1, _DHID1)),
        ],
        out_specs=[
            pl.BlockSpec((_BI, _DHID1), lambda i: (i, 0)),
            pl.BlockSpec((_BI, _DHID1), lambda i: (i, 0)),
        ],
        out_shape=[jax.ShapeDtypeStruct((_N, _DHID1), f32)] * 2,
        scratch_shapes=[pltpu.VMEM((_N, _NHID), f32)] * 2,
    )(adj_norm_pos, adj_norm_neg, x, W_pos, W_neg,
      b_pos.reshape(1, -1), b_neg.reshape(1, -1),
      W1[:_NHID], W1[_NHID:], b1.reshape(1, -1),
      W2, b2.reshape(1, -1), W3, b3.reshape(1, -1), Wd)

    y = pl.pallas_call(
        _dec_kernel,
        grid=(_N // _BD,),
        in_specs=[
            pl.BlockSpec((_BD, _DHID1), lambda i: (i, 0)),
            pl.BlockSpec((_N, _DHID1), lambda i: (0, 0)),
        ],
        out_specs=pl.BlockSpec((_BD, _N), lambda i: (i, 0)),
        out_shape=jax.ShapeDtypeStruct((_N, _N), f32),
    )(t, h)
    return y
